# Initial kernel scaffold; baseline (speedup 1.0000x reference)
import jax
import jax.numpy as jnp
from jax.experimental import pallas as pl


def kernel(probas, labels):
    raise NotImplementedError("write your pallas kernel here")


# speedup vs baseline: 43.2678x; 43.2678x over previous
import functools

import jax
import jax.numpy as jnp
from jax import lax
from jax.experimental import pallas as pl
from jax.experimental.pallas import tpu as pltpu
from jax.experimental.pallas import tpu_sc as plsc

K = 1024
LANES = 16


def _sc_histogram_kernel(C, HW, PPW, S, pf_hbm, lf_hbm, out_hbm, hist, lab_v,
                         prob_v):
    cid = lax.axis_index("c")
    sid = lax.axis_index("s")
    wid = sid * 2 + cid
    base = wid * PPW
    wpb = HW // PPW
    b = wid // wpb
    off = base - b * HW

    nhist = 2 * C * K

    def zero_body(i, _):
        hist[pl.ds(i * LANES, LANES)] = jnp.zeros((LANES,), jnp.float32)
        return 0

    lax.fori_loop(0, nhist // LANES, zero_body, 0)

    ones = jnp.ones((LANES,), jnp.float32)
    nvec = S // LANES

    def chunk_body(t, _):
        pix = base + t * S
        pltpu.sync_copy(lf_hbm.at[pl.ds(pix, S)], lab_v)
        for c in range(C):
            src = (b * C + c) * HW + off + t * S
            pltpu.sync_copy(pf_hbm.at[pl.ds(src, S)], prob_v.at[pl.ds(c * S, S)])

        def vec_body(v, _):
            lab = lab_v[pl.ds(v * LANES, LANES)]
            for c in range(C):
                p = prob_v[pl.ds(c * S + v * LANES, LANES)]
                fg = lab == c
                bb = jnp.minimum((p * float(K)).astype(jnp.int32), K - 1)
                idx = jnp.where(fg, (2 * c + 2) * K - 1 - bb, 2 * c * K + bb)
                plsc.addupdate_scatter(hist, [idx], ones)
            return 0

        lax.fori_loop(0, nvec, vec_body, 0)
        return 0

    lax.fori_loop(0, PPW // S, chunk_body, 0)
    pltpu.sync_copy(hist, out_hbm.at[pl.ds(wid * nhist, nhist)])


def _finalize_kernel(C, hist_ref, out_ref):
    h = jnp.sum(hist_ref[...], axis=0)
    h3 = h.reshape(C, 2, K)
    n1 = h3[:, 1, :]
    nall = h3[:, 0, :] + n1
    tri = (lax.broadcasted_iota(jnp.int32, (K, K), 0) >=
           lax.broadcasted_iota(jnp.int32, (K, K), 1)).astype(jnp.float32)
    N = jax.lax.dot_general(nall, tri, (((1,), (0,)), ((), ())),
                            preferred_element_type=jnp.float32)
    F = jax.lax.dot_general(n1, tri, (((1,), (0,)), ((), ())),
                            preferred_element_type=jnp.float32)
    gts = F[:, 0:1]
    denom = gts + N - F
    J = jnp.where(denom > 0, 1.0 - (gts - F) / jnp.maximum(denom, 1.0), 0.0)
    losses = (jnp.sum(J, axis=1) - 0.5 * J[:, 0]) / float(K)
    maskv = (gts[:, 0] > 0).astype(jnp.float32)
    val = jnp.sum(losses * maskv) / jnp.sum(maskv)
    out_ref[...] = val.reshape(1, 1)


def kernel(probas, labels):
    B, C, H, W = probas.shape
    HW = H * W
    P = B * HW
    NW = 32
    PPW = P // NW
    S = 2048

    pf = probas.reshape(-1)
    lf = labels.reshape(-1).astype(jnp.int32)

    mesh = plsc.VectorSubcoreMesh(core_axis_name="c", subcore_axis_name="s")
    hist = pl.kernel(
        functools.partial(_sc_histogram_kernel, C, HW, PPW, S),
        mesh=mesh,
        compiler_params=pltpu.CompilerParams(needs_layout_passes=False),
        out_type=jax.ShapeDtypeStruct((NW * 2 * C * K,), jnp.float32),
        scratch_types=[
            pltpu.VMEM((2 * C * K,), jnp.float32),
            pltpu.VMEM((S,), jnp.int32),
            pltpu.VMEM((C * S,), jnp.float32),
        ],
    )(pf, lf)

    out = pl.pallas_call(
        functools.partial(_finalize_kernel, C),
        out_shape=jax.ShapeDtypeStruct((1, 1), jnp.float32),
    )(hist.reshape(NW, 2 * C, K))
    return out.reshape(())


# speedup vs baseline: 58.4544x; 1.3510x over previous
import functools

import jax
import jax.numpy as jnp
from jax import lax
from jax.experimental import pallas as pl
from jax.experimental.pallas import tpu as pltpu
from jax.experimental.pallas import tpu_sc as plsc

K = 512
LANES = 16


def _sc_histogram_kernel(C, HW, PPW, S, pf_hbm, lf_hbm, out_hbm, hist,
                         prob_a, prob_b, lab_a, lab_b, sem_a, sem_b):
    cid = lax.axis_index("c")
    sid = lax.axis_index("s")
    wid = sid * 2 + cid
    base = wid * PPW
    wpb = HW // PPW
    b = wid // wpb
    off = base - b * HW

    nhist = 2 * C * K
    CS = C * S
    row0 = b * C * HW + off

    def zero_body(i, _):
        hist[pl.ds(i * LANES, LANES)] = jnp.zeros((LANES,), jnp.float32)
        return 0

    lax.fori_loop(0, nhist // LANES, zero_body, 0)

    def issue(t, prob_v, lab_v, sem):
        pltpu.async_copy(lf_hbm.at[pl.ds(base + t * S, S)], lab_v, sem)
        for c in range(C):
            pltpu.async_copy(pf_hbm.at[pl.ds(row0 + c * HW + t * S, S)],
                             prob_v.at[pl.ds(c * S, S)], sem)

    def drain(prob_v, lab_v, sem):
        pltpu.make_async_copy(lf_hbm.at[pl.ds(0, S)], lab_v, sem).wait()
        pltpu.make_async_copy(pf_hbm.at[pl.ds(0, CS)], prob_v, sem).wait()

    ones = jnp.ones((LANES,), jnp.float32)
    nvec = S // LANES

    def compute(prob_v, lab_v):
        def vec_body(v, _):
            lab = lab_v[pl.ds(v * LANES, LANES)]
            for c in range(C):
                p = prob_v[pl.ds(c * S + v * LANES, LANES)]
                fg = lab == c
                bb = jnp.minimum((p * float(K)).astype(jnp.int32), K - 1)
                idx = jnp.where(fg, (2 * c + 2) * K - 1 - bb, 2 * c * K + bb)
                plsc.addupdate_scatter(hist, [idx], ones)
            return 0

        lax.fori_loop(0, nvec, vec_body, 0)

    NCH = PPW // S
    issue(0, prob_a, lab_a, sem_a)

    def pair_body(i, _):
        t0 = 2 * i
        issue(t0 + 1, prob_b, lab_b, sem_b)
        drain(prob_a, lab_a, sem_a)
        compute(prob_a, lab_a)

        @pl.when(t0 + 2 < NCH)
        def _prefetch():
            issue(t0 + 2, prob_a, lab_a, sem_a)

        drain(prob_b, lab_b, sem_b)
        compute(prob_b, lab_b)
        return 0

    lax.fori_loop(0, NCH // 2, pair_body, 0)
    pltpu.sync_copy(hist, out_hbm.at[pl.ds(wid * nhist, nhist)])


def _finalize_kernel(C, hist_ref, out_ref):
    h = jnp.sum(hist_ref[...], axis=0)
    h3 = h.reshape(C, 2, K)
    n1 = h3[:, 1, :]
    nall = h3[:, 0, :] + n1
    tri = (lax.broadcasted_iota(jnp.int32, (K, K), 0) >=
           lax.broadcasted_iota(jnp.int32, (K, K), 1)).astype(jnp.float32)
    N = jax.lax.dot_general(nall, tri, (((1,), (0,)), ((), ())),
                            preferred_element_type=jnp.float32)
    F = jax.lax.dot_general(n1, tri, (((1,), (0,)), ((), ())),
                            preferred_element_type=jnp.float32)
    gts = F[:, 0:1]
    denom = gts + N - F
    J = jnp.where(denom > 0, 1.0 - (gts - F) / jnp.maximum(denom, 1.0), 0.0)
    losses = (jnp.sum(J, axis=1) - 0.5 * J[:, 0]) / float(K)
    maskv = (gts[:, 0] > 0).astype(jnp.float32)
    val = jnp.sum(losses * maskv) / jnp.sum(maskv)
    out_ref[...] = val.reshape(1, 1)


def kernel(probas, labels):
    B, C, H, W = probas.shape
    HW = H * W
    P = B * HW
    NW = 32
    PPW = P // NW
    S = 2048

    pf = probas.reshape(-1)
    lf = labels.reshape(-1).astype(jnp.int32)

    mesh = plsc.VectorSubcoreMesh(core_axis_name="c", subcore_axis_name="s")
    hist = pl.kernel(
        functools.partial(_sc_histogram_kernel, C, HW, PPW, S),
        mesh=mesh,
        compiler_params=pltpu.CompilerParams(needs_layout_passes=False),
        out_type=jax.ShapeDtypeStruct((NW * 2 * C * K,), jnp.float32),
        scratch_types=[
            pltpu.VMEM((2 * C * K,), jnp.float32),
            pltpu.VMEM((C * S,), jnp.float32),
            pltpu.VMEM((C * S,), jnp.float32),
            pltpu.VMEM((S,), jnp.int32),
            pltpu.VMEM((S,), jnp.int32),
            pltpu.SemaphoreType.DMA,
            pltpu.SemaphoreType.DMA,
        ],
    )(pf, lf)

    out = pl.pallas_call(
        functools.partial(_finalize_kernel, C),
        out_shape=jax.ShapeDtypeStruct((1, 1), jnp.float32),
    )(hist.reshape(NW, 2 * C, K))
    return out.reshape(())
